# baseline (speedup 1.0000x reference)
import functools

import jax
import jax.numpy as jnp
from jax import lax
from jax.experimental import pallas as pl
from jax.experimental.pallas import tpu as pltpu
from jax.experimental.pallas import tpu_sc as plsc

N = 10000
E = 320000
D_IN = 128
D_HID = 128
N_CLS = 40

NC, NS = 2, 16
NW = NC * NS
EPW = E // NW
K = 80
NBLK = EPW // K
KA = 128
NBLKA = EPW // KA
REM = EPW - NBLKA * KA
NBUF = 8
NP = 10240
RPT = NP // NS
DD = 16


def _sc_mesh():
    return plsc.VectorSubcoreMesh(
        core_axis_name="c", subcore_axis_name="s", num_cores=NC, num_subcores=NS
    )


def _make_agg(D):
    @functools.partial(
        pl.kernel,
        out_type=jax.ShapeDtypeStruct((2 * NP, D), jnp.float32),
        mesh=_sc_mesh(),
        compiler_params=pltpu.CompilerParams(use_tc_tiling_on_sc=False),
        scratch_types=[
            pltpu.VMEM((EPW,), jnp.int32),
            pltpu.VMEM((EPW,), jnp.int32),
            [pltpu.VMEM((KA, D), jnp.float32) for _ in range(NBUF)],
            pltpu.VMEM_SHARED((NP, D), jnp.float32),
            [pltpu.SemaphoreType.DMA for _ in range(NBUF)],
            [pltpu.SemaphoreType.DMA for _ in range(NBUF)],
        ],
    )
    def agg(h_hbm, edge_hbm, zeros_hbm, out_hbm,
            src_v, dst_v, rows, acc, gsem, ssem):
        c = lax.axis_index("c")
        s = lax.axis_index("s")
        wid = s * NC + c
        pltpu.sync_copy(zeros_hbm, acc.at[pl.ds(s * RPT, RPT)])
        pltpu.sync_copy(edge_hbm.at[0, pl.ds(wid * EPW, EPW)], src_v)
        pltpu.sync_copy(edge_hbm.at[1, pl.ds(wid * EPW, EPW)], dst_v)
        plsc.subcore_barrier()

        def issue_gather(j, b):
            pltpu.async_copy(h_hbm.at[src_v.at[pl.ds(j * KA, KA)]], rows[b],
                             gsem[b])

        def wait_gather(j, b):
            pltpu.make_async_copy(h_hbm.at[src_v.at[pl.ds(j * KA, KA)]],
                                  rows[b], gsem[b]).wait()

        def issue_scatter(j, b):
            pltpu.async_copy(rows[b], acc.at[dst_v.at[pl.ds(j * KA, KA)]],
                             ssem[b], add=True)

        def wait_scatter(j, b):
            pltpu.make_async_copy(rows[b], acc.at[dst_v.at[pl.ds(j * KA, KA)]],
                                  ssem[b]).wait()

        n_main = (NBLKA - NBUF) // NBUF
        tail_start = 1 + n_main * NBUF

        for b in range(NBUF - 1):
            issue_gather(b, b)
        wait_gather(0, 0)
        issue_scatter(0, 0)
        issue_gather(NBUF - 1, NBUF - 1)

        def body(i, carry):
            for b in range(NBUF):
                j = i * NBUF + 1 + b
                cb = (b + 1) % NBUF
                wait_gather(j, cb)
                issue_scatter(j, cb)
                wait_scatter(j - 1, b % NBUF)
                issue_gather(j + NBUF - 1, b % NBUF)
            return carry

        lax.fori_loop(0, n_main, body, 0)

        for j in range(tail_start, NBLKA):
            cb = j % NBUF
            wait_gather(j, cb)
            issue_scatter(j, cb)
            wait_scatter(j - 1, (j - 1) % NBUF)
            if j + NBUF - 1 < NBLKA:
                issue_gather(j + NBUF - 1, (j - 1) % NBUF)
        wait_scatter(NBLKA - 1, (NBLKA - 1) % NBUF)

        if REM:
            rb = rows[0].at[pl.ds(0, REM)]
            ri = pl.ds(NBLKA * KA, REM)
            pltpu.async_copy(h_hbm.at[src_v.at[ri]], rb, gsem[0]).wait()
            pltpu.sync_copy(rb, acc.at[dst_v.at[ri]], add=True)

        plsc.subcore_barrier()
        pltpu.sync_copy(acc.at[pl.ds(s * RPT, RPT)],
                        out_hbm.at[pl.ds(c * NP + s * RPT, RPT)])

    return agg


def _make_deg():
    @functools.partial(
        pl.kernel,
        out_type=jax.ShapeDtypeStruct((2 * NP, DD), jnp.float32),
        mesh=_sc_mesh(),
        compiler_params=pltpu.CompilerParams(use_tc_tiling_on_sc=False),
        scratch_types=[
            pltpu.VMEM((EPW,), jnp.int32),
            pltpu.VMEM((K, DD), jnp.float32),
            pltpu.VMEM_SHARED((NP, DD), jnp.float32),
        ],
    )
    def deg(edge_hbm, zeros_hbm, ones_hbm, out_hbm, dst_v, ones_v, acc):
        c = lax.axis_index("c")
        s = lax.axis_index("s")
        wid = s * NC + c
        pltpu.sync_copy(zeros_hbm, acc.at[pl.ds(s * RPT, RPT)])
        pltpu.sync_copy(ones_hbm, ones_v)
        pltpu.sync_copy(edge_hbm.at[1, pl.ds(wid * EPW, EPW)], dst_v)
        plsc.subcore_barrier()

        def body(j, carry):
            pltpu.sync_copy(ones_v, acc.at[dst_v.at[pl.ds(j * K, K)]],
                            add=True)
            return carry

        lax.fori_loop(0, NBLK, body, 0)
        plsc.subcore_barrier()
        pltpu.sync_copy(acc.at[pl.ds(s * RPT, RPT)],
                        out_hbm.at[pl.ds(c * NP + s * RPT, RPT)])

    return deg


DH = 64
_agg64 = _make_agg(DH)
_deg = _make_deg()

def _tc1_body(x_ref, w_ref, degp_ref, hlo_ref, hhi_ref, dis_ref):
    d = degp_ref[0:N, 0:1] + degp_ref[NP:NP + N, 0:1] + 1.0
    dis = lax.rsqrt(d)
    h = jnp.dot(x_ref[...], w_ref[...], preferred_element_type=jnp.float32)
    h = h * dis
    hlo_ref[...] = h[:, :DH]
    hhi_ref[...] = h[:, DH:]
    dis_ref[...] = dis


def _tc2_body(alo_ref, ahi_ref, hlo_ref, hhi_ref, dis_ref, b1_ref, w_ref,
              out_ref):
    dis = dis_ref[...]
    lo = alo_ref[0:N] + alo_ref[NP:NP + N] + hlo_ref[...]
    hi = ahi_ref[0:N] + ahi_ref[NP:NP + N] + hhi_ref[...]
    t = jnp.concatenate([lo, hi], axis=1) * dis + b1_ref[...]
    t = jnp.maximum(t, 0.0)
    out_ref[...] = jnp.dot(t, w_ref[...], preferred_element_type=jnp.float32) * dis


def _tc3_body(acc_ref, h2_ref, dis_ref, b2_ref, out_ref):
    t = acc_ref[0:N] + acc_ref[NP:NP + N] + h2_ref[...]
    out_ref[...] = t[:, :N_CLS] * dis_ref[...] + b2_ref[...]


def kernel(x, edge_index, W1, b1, W2, b2):
    f32 = jnp.float32
    edges = edge_index.astype(jnp.int32)
    z16 = jnp.zeros((RPT, DD), f32)
    z64 = jnp.zeros((RPT, DH), f32)
    ones16 = jnp.ones((K, DD), f32)
    W2p = jnp.pad(W2, ((0, 0), (0, DH - N_CLS)))

    degp = _deg(edges, z16, ones16)

    h1lo, h1hi, dis = pl.pallas_call(
        _tc1_body,
        out_shape=[
            jax.ShapeDtypeStruct((N, DH), f32),
            jax.ShapeDtypeStruct((N, DH), f32),
            jax.ShapeDtypeStruct((N, 1), f32),
        ],
    )(x, W1, degp)

    a1lo = _agg64(h1lo, edges, z64)
    a1hi = _agg64(h1hi, edges, z64)

    h2p = pl.pallas_call(
        _tc2_body,
        out_shape=jax.ShapeDtypeStruct((N, DH), f32),
    )(a1lo, a1hi, h1lo, h1hi, dis, b1.reshape(1, D_HID), W2p)

    a2 = _agg64(h2p, edges, z64)

    out = pl.pallas_call(
        _tc3_body,
        out_shape=jax.ShapeDtypeStruct((N, N_CLS), f32),
    )(a2, h2p, dis, b2.reshape(1, N_CLS))

    return out


# speedup vs baseline: 1.0168x; 1.0168x over previous
import functools

import jax
import jax.numpy as jnp
from jax import lax
from jax.experimental import pallas as pl
from jax.experimental.pallas import tpu as pltpu
from jax.experimental.pallas import tpu_sc as plsc

N = 10000
E = 320000
D_IN = 128
D_HID = 128
N_CLS = 40

NC, NS = 2, 16
NW = NC * NS
EPW = E // NW
K = 80
NBLK = EPW // K
KA = 128
NBLKA = EPW // KA
REM = EPW - NBLKA * KA
NBUF = 6
NP = 10240
RPT = NP // NS
DD = 16


def _sc_mesh():
    return plsc.VectorSubcoreMesh(
        core_axis_name="c", subcore_axis_name="s", num_cores=NC, num_subcores=NS
    )


def _make_agg(D):
    @functools.partial(
        pl.kernel,
        out_type=jax.ShapeDtypeStruct((2 * NP, D), jnp.float32),
        mesh=_sc_mesh(),
        compiler_params=pltpu.CompilerParams(use_tc_tiling_on_sc=False),
        scratch_types=[
            pltpu.VMEM((EPW,), jnp.int32),
            pltpu.VMEM((EPW,), jnp.int32),
            [pltpu.VMEM((KA, D), jnp.float32) for _ in range(NBUF)],
            pltpu.VMEM_SHARED((NP, D), jnp.float32),
            [pltpu.SemaphoreType.DMA for _ in range(NBUF)],
            [pltpu.SemaphoreType.DMA for _ in range(NBUF)],
        ],
    )
    def agg(h_hbm, edge_hbm, zeros_hbm, out_hbm,
            src_v, dst_v, rows, acc, gsem, ssem):
        c = lax.axis_index("c")
        s = lax.axis_index("s")
        wid = s * NC + c
        pltpu.sync_copy(zeros_hbm, acc.at[pl.ds(s * RPT, RPT)])
        pltpu.sync_copy(edge_hbm.at[0, pl.ds(wid * EPW, EPW)], src_v)
        pltpu.sync_copy(edge_hbm.at[1, pl.ds(wid * EPW, EPW)], dst_v)
        plsc.subcore_barrier()

        def issue_gather(j, b):
            pltpu.async_copy(h_hbm.at[src_v.at[pl.ds(j * KA, KA)]], rows[b],
                             gsem[b])

        def wait_gather(j, b):
            pltpu.make_async_copy(h_hbm.at[src_v.at[pl.ds(j * KA, KA)]],
                                  rows[b], gsem[b]).wait()

        def issue_scatter(j, b):
            pltpu.async_copy(rows[b], acc.at[dst_v.at[pl.ds(j * KA, KA)]],
                             ssem[b], add=True)

        def wait_scatter(j, b):
            pltpu.make_async_copy(rows[b], acc.at[dst_v.at[pl.ds(j * KA, KA)]],
                                  ssem[b]).wait()

        n_main = (NBLKA - NBUF) // NBUF
        tail_start = 1 + n_main * NBUF

        for b in range(NBUF - 1):
            issue_gather(b, b)
        wait_gather(0, 0)
        issue_scatter(0, 0)
        issue_gather(NBUF - 1, NBUF - 1)

        def body(i, carry):
            for b in range(NBUF):
                j = i * NBUF + 1 + b
                cb = (b + 1) % NBUF
                wait_gather(j, cb)
                issue_scatter(j, cb)
                wait_scatter(j - 1, b % NBUF)
                issue_gather(j + NBUF - 1, b % NBUF)
            return carry

        lax.fori_loop(0, n_main, body, 0)

        for j in range(tail_start, NBLKA):
            cb = j % NBUF
            wait_gather(j, cb)
            issue_scatter(j, cb)
            wait_scatter(j - 1, (j - 1) % NBUF)
            if j + NBUF - 1 < NBLKA:
                issue_gather(j + NBUF - 1, (j - 1) % NBUF)
        wait_scatter(NBLKA - 1, (NBLKA - 1) % NBUF)

        if REM:
            rb = rows[0].at[pl.ds(0, REM)]
            ri = pl.ds(NBLKA * KA, REM)
            pltpu.async_copy(h_hbm.at[src_v.at[ri]], rb, gsem[0]).wait()
            pltpu.sync_copy(rb, acc.at[dst_v.at[ri]], add=True)

        plsc.subcore_barrier()
        pltpu.sync_copy(acc.at[pl.ds(s * RPT, RPT)],
                        out_hbm.at[pl.ds(c * NP + s * RPT, RPT)])

    return agg


def _make_deg():
    @functools.partial(
        pl.kernel,
        out_type=jax.ShapeDtypeStruct((2 * NP, DD), jnp.float32),
        mesh=_sc_mesh(),
        compiler_params=pltpu.CompilerParams(use_tc_tiling_on_sc=False),
        scratch_types=[
            pltpu.VMEM((EPW,), jnp.int32),
            pltpu.VMEM((K, DD), jnp.float32),
            pltpu.VMEM_SHARED((NP, DD), jnp.float32),
            pltpu.SemaphoreType.DMA,
        ],
    )
    def deg(edge_hbm, zeros_hbm, ones_hbm, out_hbm, dst_v, ones_v, acc, sem):
        c = lax.axis_index("c")
        s = lax.axis_index("s")
        wid = s * NC + c
        pltpu.sync_copy(zeros_hbm, acc.at[pl.ds(s * RPT, RPT)])
        pltpu.sync_copy(ones_hbm, ones_v)
        pltpu.sync_copy(edge_hbm.at[1, pl.ds(wid * EPW, EPW)], dst_v)
        plsc.subcore_barrier()

        def body(j, carry):
            pltpu.async_copy(ones_v, acc.at[dst_v.at[pl.ds(j * K, K)]],
                             sem, add=True)
            return carry

        lax.fori_loop(0, NBLK, body, 0)

        def drain(j, carry):
            pltpu.make_async_copy(
                ones_v, acc.at[dst_v.at[pl.ds(j * K, K)]], sem).wait()
            return carry

        lax.fori_loop(0, NBLK, drain, 0)
        plsc.subcore_barrier()
        pltpu.sync_copy(acc.at[pl.ds(s * RPT, RPT)],
                        out_hbm.at[pl.ds(c * NP + s * RPT, RPT)])

    return deg


DH = 64
_agg64 = _make_agg(DH)
_deg = _make_deg()

def _tc1a_body(x_ref, w_ref, h_ref):
    h_ref[...] = jnp.dot(x_ref[...], w_ref[...],
                         preferred_element_type=jnp.float32)


def _tc1b_body(h_ref, degp_ref, hlo_ref, hhi_ref, dis_ref):
    d = degp_ref[0:N, 0:1] + degp_ref[NP:NP + N, 0:1] + 1.0
    dis = lax.rsqrt(d)
    h = h_ref[...] * dis
    hlo_ref[...] = h[:, :DH]
    hhi_ref[...] = h[:, DH:]
    dis_ref[...] = dis


def _tc2_body(alo_ref, ahi_ref, hlo_ref, hhi_ref, dis_ref, b1_ref, w_ref,
              out_ref):
    dis = dis_ref[...]
    lo = alo_ref[0:N] + alo_ref[NP:NP + N] + hlo_ref[...]
    hi = ahi_ref[0:N] + ahi_ref[NP:NP + N] + hhi_ref[...]
    t = jnp.concatenate([lo, hi], axis=1) * dis + b1_ref[...]
    t = jnp.maximum(t, 0.0)
    out_ref[...] = jnp.dot(t, w_ref[...], preferred_element_type=jnp.float32) * dis


def _tc3_body(acc_ref, h2_ref, dis_ref, b2_ref, out_ref):
    t = acc_ref[0:N] + acc_ref[NP:NP + N] + h2_ref[...]
    out_ref[...] = t[:, :N_CLS] * dis_ref[...] + b2_ref[...]


def kernel(x, edge_index, W1, b1, W2, b2):
    f32 = jnp.float32
    edges = edge_index.astype(jnp.int32)
    z16 = jnp.zeros((RPT, DD), f32)
    z64 = jnp.zeros((RPT, DH), f32)
    ones16 = jnp.ones((K, DD), f32)
    W2p = jnp.pad(W2, ((0, 0), (0, DH - N_CLS)))

    h1 = pl.pallas_call(
        _tc1a_body,
        out_shape=jax.ShapeDtypeStruct((N, D_HID), f32),
    )(x, W1)

    degp = _deg(edges, z16, ones16)

    h1lo, h1hi, dis = pl.pallas_call(
        _tc1b_body,
        out_shape=[
            jax.ShapeDtypeStruct((N, DH), f32),
            jax.ShapeDtypeStruct((N, DH), f32),
            jax.ShapeDtypeStruct((N, 1), f32),
        ],
    )(h1, degp)

    a1lo = _agg64(h1lo, edges, z64)
    a1hi = _agg64(h1hi, edges, z64)

    h2p = pl.pallas_call(
        _tc2_body,
        out_shape=jax.ShapeDtypeStruct((N, DH), f32),
    )(a1lo, a1hi, h1lo, h1hi, dis, b1.reshape(1, D_HID), W2p)

    a2 = _agg64(h2p, edges, z64)

    out = pl.pallas_call(
        _tc3_body,
        out_shape=jax.ShapeDtypeStruct((N, N_CLS), f32),
    )(a2, h2p, dis, b2.reshape(1, N_CLS))

    return out


# speedup vs baseline: 1.0239x; 1.0070x over previous
import functools

import jax
import jax.numpy as jnp
from jax import lax
from jax.experimental import pallas as pl
from jax.experimental.pallas import tpu as pltpu
from jax.experimental.pallas import tpu_sc as plsc

N = 10000
E = 320000
D_IN = 128
D_HID = 128
N_CLS = 40

NC, NS = 2, 16
NW = NC * NS
EPW = E // NW
K = 80
NBLK = EPW // K
KA = 128
NBLKA = EPW // KA
REM = EPW - NBLKA * KA
NBUF = 6
NP = N
RPT = 624
REXT = NP - NS * RPT
DD = 16


def _sc_mesh():
    return plsc.VectorSubcoreMesh(
        core_axis_name="c", subcore_axis_name="s", num_cores=NC, num_subcores=NS
    )


def _make_agg(D):
    @functools.partial(
        pl.kernel,
        out_type=jax.ShapeDtypeStruct((2 * NP, D), jnp.float32),
        mesh=_sc_mesh(),
        compiler_params=pltpu.CompilerParams(use_tc_tiling_on_sc=False),
        scratch_types=[
            pltpu.VMEM((EPW,), jnp.int32),
            pltpu.VMEM((EPW,), jnp.int32),
            [pltpu.VMEM((KA, D), jnp.float32) for _ in range(NBUF)],
            pltpu.VMEM_SHARED((NP, D), jnp.float32),
            [pltpu.SemaphoreType.DMA for _ in range(NBUF)],
            [pltpu.SemaphoreType.DMA for _ in range(NBUF)],
        ],
    )
    def agg(h_hbm, edge_hbm, zeros_hbm, out_hbm,
            src_v, dst_v, rows, acc, gsem, ssem):
        c = lax.axis_index("c")
        s = lax.axis_index("s")
        wid = s * NC + c
        pltpu.sync_copy(zeros_hbm, acc.at[pl.ds(s * RPT, RPT)])

        @pl.when(s == NS - 1)
        def _():
            pltpu.sync_copy(zeros_hbm.at[pl.ds(0, REXT)],
                            acc.at[pl.ds(NS * RPT, REXT)])

        pltpu.sync_copy(edge_hbm.at[0, pl.ds(wid * EPW, EPW)], src_v)
        pltpu.sync_copy(edge_hbm.at[1, pl.ds(wid * EPW, EPW)], dst_v)
        plsc.subcore_barrier()

        def issue_gather(j, b):
            pltpu.async_copy(h_hbm.at[src_v.at[pl.ds(j * KA, KA)]], rows[b],
                             gsem[b])

        def wait_gather(j, b):
            pltpu.make_async_copy(h_hbm.at[src_v.at[pl.ds(j * KA, KA)]],
                                  rows[b], gsem[b]).wait()

        def issue_scatter(j, b):
            pltpu.async_copy(rows[b], acc.at[dst_v.at[pl.ds(j * KA, KA)]],
                             ssem[b], add=True)

        def wait_scatter(j, b):
            pltpu.make_async_copy(rows[b], acc.at[dst_v.at[pl.ds(j * KA, KA)]],
                                  ssem[b]).wait()

        n_main = (NBLKA - NBUF) // NBUF
        tail_start = 1 + n_main * NBUF

        for b in range(NBUF - 1):
            issue_gather(b, b)
        wait_gather(0, 0)
        issue_scatter(0, 0)
        issue_gather(NBUF - 1, NBUF - 1)

        def body(i, carry):
            for b in range(NBUF):
                j = i * NBUF + 1 + b
                cb = (b + 1) % NBUF
                wait_gather(j, cb)
                issue_scatter(j, cb)
                wait_scatter(j - 1, b % NBUF)
                issue_gather(j + NBUF - 1, b % NBUF)
            return carry

        lax.fori_loop(0, n_main, body, 0)

        for j in range(tail_start, NBLKA):
            cb = j % NBUF
            wait_gather(j, cb)
            issue_scatter(j, cb)
            wait_scatter(j - 1, (j - 1) % NBUF)
            if j + NBUF - 1 < NBLKA:
                issue_gather(j + NBUF - 1, (j - 1) % NBUF)
        wait_scatter(NBLKA - 1, (NBLKA - 1) % NBUF)

        if REM:
            rb = rows[0].at[pl.ds(0, REM)]
            ri = pl.ds(NBLKA * KA, REM)
            pltpu.async_copy(h_hbm.at[src_v.at[ri]], rb, gsem[0]).wait()
            pltpu.sync_copy(rb, acc.at[dst_v.at[ri]], add=True)

        plsc.subcore_barrier()
        pltpu.sync_copy(acc.at[pl.ds(s * RPT, RPT)],
                        out_hbm.at[pl.ds(c * NP + s * RPT, RPT)])

        @pl.when(s == NS - 1)
        def _():
            pltpu.sync_copy(acc.at[pl.ds(NS * RPT, REXT)],
                            out_hbm.at[pl.ds(c * NP + NS * RPT, REXT)])

    return agg


def _make_deg():
    @functools.partial(
        pl.kernel,
        out_type=jax.ShapeDtypeStruct((2 * NP, DD), jnp.float32),
        mesh=_sc_mesh(),
        compiler_params=pltpu.CompilerParams(use_tc_tiling_on_sc=False),
        scratch_types=[
            pltpu.VMEM((EPW,), jnp.int32),
            pltpu.VMEM((K, DD), jnp.float32),
            pltpu.VMEM_SHARED((NP, DD), jnp.float32),
            pltpu.SemaphoreType.DMA,
        ],
    )
    def deg(edge_hbm, zeros_hbm, ones_hbm, out_hbm, dst_v, ones_v, acc, sem):
        c = lax.axis_index("c")
        s = lax.axis_index("s")
        wid = s * NC + c
        pltpu.sync_copy(zeros_hbm, acc.at[pl.ds(s * RPT, RPT)])

        @pl.when(s == NS - 1)
        def _():
            pltpu.sync_copy(zeros_hbm.at[pl.ds(0, REXT)],
                            acc.at[pl.ds(NS * RPT, REXT)])

        pltpu.sync_copy(ones_hbm, ones_v)
        pltpu.sync_copy(edge_hbm.at[1, pl.ds(wid * EPW, EPW)], dst_v)
        plsc.subcore_barrier()

        def body(j, carry):
            pltpu.async_copy(ones_v, acc.at[dst_v.at[pl.ds(j * K, K)]],
                             sem, add=True)
            return carry

        lax.fori_loop(0, NBLK, body, 0)

        def drain(j, carry):
            pltpu.make_async_copy(
                ones_v, acc.at[dst_v.at[pl.ds(j * K, K)]], sem).wait()
            return carry

        lax.fori_loop(0, NBLK, drain, 0)
        plsc.subcore_barrier()
        pltpu.sync_copy(acc.at[pl.ds(s * RPT, RPT)],
                        out_hbm.at[pl.ds(c * NP + s * RPT, RPT)])

        @pl.when(s == NS - 1)
        def _():
            pltpu.sync_copy(acc.at[pl.ds(NS * RPT, REXT)],
                            out_hbm.at[pl.ds(c * NP + NS * RPT, REXT)])

    return deg


DH = 64
_agg64 = _make_agg(DH)
_deg = _make_deg()

_R = 2000
_GB = N // _R


def _tc1a_body(x_ref, w_ref, h_ref):
    h_ref[...] = jnp.dot(x_ref[...], w_ref[...],
                         preferred_element_type=jnp.float32)


def _tc1b_body(h_ref, dega_ref, degb_ref, hlo_ref, hhi_ref, dis_ref):
    d = dega_ref[:, 0:1] + degb_ref[:, 0:1] + 1.0
    dis = lax.rsqrt(d)
    h = h_ref[...] * dis
    hlo_ref[...] = h[:, :DH]
    hhi_ref[...] = h[:, DH:]
    dis_ref[...] = dis


def _tc2_body(aloa_ref, alob_ref, ahia_ref, ahib_ref, hlo_ref, hhi_ref,
              dis_ref, b1_ref, w_ref, out_ref):
    dis = dis_ref[...]
    lo = aloa_ref[...] + alob_ref[...] + hlo_ref[...]
    hi = ahia_ref[...] + ahib_ref[...] + hhi_ref[...]
    t = jnp.concatenate([lo, hi], axis=1) * dis + b1_ref[...]
    t = jnp.maximum(t, 0.0)
    out_ref[...] = jnp.dot(t, w_ref[...], preferred_element_type=jnp.float32) * dis


def _tc3_body(acca_ref, accb_ref, h2_ref, dis_ref, b2_ref, out_ref):
    t = acca_ref[...] + accb_ref[...] + h2_ref[...]
    out_ref[...] = t[:, :N_CLS] * dis_ref[...] + b2_ref[...]


def _rows(d):
    return pl.BlockSpec((_R, d), lambda i: (i, 0))


def _rows2(d):
    return pl.BlockSpec((_R, d), lambda i: (_GB + i, 0))


def _full(a, b):
    return pl.BlockSpec((a, b), lambda i: (0, 0))


def kernel(x, edge_index, W1, b1, W2, b2):
    f32 = jnp.float32
    edges = edge_index.astype(jnp.int32)
    z16 = jnp.zeros((RPT, DD), f32)
    z64 = jnp.zeros((RPT, DH), f32)
    ones16 = jnp.ones((K, DD), f32)
    W2p = jnp.pad(W2, ((0, 0), (0, DH - N_CLS)))

    grid = (_GB,)
    h1 = pl.pallas_call(
        _tc1a_body,
        grid=grid,
        in_specs=[_rows(D_IN), _full(D_IN, D_HID)],
        out_specs=_rows(D_HID),
        out_shape=jax.ShapeDtypeStruct((N, D_HID), f32),
    )(x, W1)

    degp = _deg(edges, z16, ones16)

    h1lo, h1hi, dis = pl.pallas_call(
        _tc1b_body,
        grid=grid,
        in_specs=[_rows(D_HID), _rows(DD), _rows2(DD)],
        out_specs=[_rows(DH), _rows(DH), _rows(1)],
        out_shape=[
            jax.ShapeDtypeStruct((N, DH), f32),
            jax.ShapeDtypeStruct((N, DH), f32),
            jax.ShapeDtypeStruct((N, 1), f32),
        ],
    )(h1, degp, degp)

    a1lo = _agg64(h1lo, edges, z64)
    a1hi = _agg64(h1hi, edges, z64)

    h2p = pl.pallas_call(
        _tc2_body,
        grid=grid,
        in_specs=[_rows(DH), _rows2(DH), _rows(DH), _rows2(DH),
                  _rows(DH), _rows(DH), _rows(1),
                  _full(1, D_HID), _full(D_HID, DH)],
        out_specs=_rows(DH),
        out_shape=jax.ShapeDtypeStruct((N, DH), f32),
    )(a1lo, a1lo, a1hi, a1hi, h1lo, h1hi, dis,
      b1.reshape(1, D_HID), W2p)

    a2 = _agg64(h2p, edges, z64)

    out = pl.pallas_call(
        _tc3_body,
        grid=grid,
        in_specs=[_rows(DH), _rows2(DH), _rows(DH), _rows(1),
                  _full(1, N_CLS)],
        out_specs=_rows(N_CLS),
        out_shape=jax.ShapeDtypeStruct((N, N_CLS), f32),
    )(a2, a2, h2p, dis, b2.reshape(1, N_CLS))

    return out
